# Initial kernel scaffold; baseline (speedup 1.0000x reference)
import jax
import jax.numpy as jnp
from jax.experimental import pallas as pl


def kernel(cell, x, z, struct_size, emb, Wm0, bm0, Wm1, bm1, Wm2, bm2, Wu0, bu0, Wu1, bu1, Wu2, bu2, mW0, mb0, mW1, mb1, mW2, mb2, mW3, mb3, mW4, mb4):
    raise NotImplementedError("write your pallas kernel here")


# speedup vs baseline: 9.6075x; 9.6075x over previous
import jax
import jax.numpy as jnp
from jax.experimental import pallas as pl

_B = 64
_NPER = 128
_N = _B * _NPER
_F = 256
_KNN = 16
_LAYERS = 3
_NZ = 100

_PREC = jax.lax.Precision.HIGHEST


def _dot(a, b):
    return jnp.dot(a, b, preferred_element_type=jnp.float32, precision=_PREC)


def _mpnn_body(xc_ref, xr_ref, z_ref, inv_ref, emb_ref, Wm_ref, bm_ref,
               Wu_ref, bu_ref, lat_ref):
    xc = xc_ref[0]
    xr = xr_ref[0]
    fc = jnp.mod(xc, 1.0)
    fr = jnp.mod(xr, 1.0)

    ri = jax.lax.broadcasted_iota(jnp.int32, (_NPER, _NPER), 0)
    ci = jax.lax.broadcasted_iota(jnp.int32, (_NPER, _NPER), 1)
    dist2 = jnp.where(ri == ci, 1e9, 0.0).astype(jnp.float32)
    for c in range(3):
        dc = fc[:, c:c + 1] - fr[c:c + 1, :]
        dc = dc - jnp.round(dc)
        dist2 = dist2 + dc * dc

    idx_cols = []
    dists = []
    for _ in range(_KNN):
        m = jnp.min(dist2, axis=1, keepdims=True)
        cand = jnp.where(dist2 == m, ci, _NPER)
        a = jnp.min(cand, axis=1, keepdims=True)
        idx_cols.append(a)
        dists.append(jnp.sqrt(m + 1e-12))
        dist2 = jnp.where(ci == a, 3e9, dist2)

    zc = z_ref[0]
    zoh = (zc == jax.lax.broadcasted_iota(jnp.int32, (_NPER, _NZ), 1)
           ).astype(jnp.float32)
    h = _dot(zoh, emb_ref[...])

    for l in range(_LAYERS):
        Wm = Wm_ref[l]
        Adst = _dot(h, Wm[0:_F]) + bm_ref[l]
        Asrc = _dot(h, Wm[_F:2 * _F])
        wd = Wm[2 * _F:2 * _F + 1]
        acc = jnp.zeros((_NPER, _F), dtype=jnp.float32)
        for k in range(_KNN):
            oh = (idx_cols[k] == ci).astype(jnp.float32)
            g = _dot(oh, Asrc)
            acc = acc + jax.nn.relu(g + Adst + dists[k] * wd)
        agg = acc * (1.0 / _KNN)
        Wu = Wu_ref[l]
        u = _dot(h, Wu[0:_F]) + _dot(agg, Wu[_F:2 * _F]) + bu_ref[l]
        h = h + jax.nn.relu(u)

    lat_ref[0] = jnp.sum(h, axis=0, keepdims=True) * inv_ref[0]


def _mlp_body(lat_ref, w0, b0, w1, b1, w2, b2, w3, b3, w4, b4, out_ref):
    t = lat_ref[...]
    for i, (w, b) in enumerate(((w0, b0), (w1, b1), (w2, b2),
                                (w3, b3), (w4, b4))):
        t = _dot(t, w[...]) + b[...]
        if i < 4:
            t = jax.nn.relu(t)
    out_ref[...] = t


def kernel(cell, x, z, struct_size, emb, Wm0, bm0, Wm1, bm1, Wm2, bm2,
           Wu0, bu0, Wu1, bu1, Wu2, bu2, mW0, mb0, mW1, mb1, mW2, mb2,
           mW3, mb3, mW4, mb4):
    f32 = jnp.float32
    xb = x.reshape(_B, _NPER, 3)
    xr = xb.transpose(0, 2, 1)
    z3 = z.reshape(_B, _NPER, 1)
    inv = (1.0 / struct_size.astype(f32)).reshape(_B, 1, 1)
    Wm = jnp.stack([Wm0, Wm1, Wm2])
    bm = jnp.stack([bm0, bm1, bm2]).reshape(_LAYERS, 1, _F)
    Wu = jnp.stack([Wu0, Wu1, Wu2])
    bu = jnp.stack([bu0, bu1, bu2]).reshape(_LAYERS, 1, _F)

    full = lambda shp: pl.BlockSpec(shp, lambda s: (0,) * len(shp))
    lat = pl.pallas_call(
        _mpnn_body,
        grid=(_B,),
        in_specs=[
            pl.BlockSpec((1, _NPER, 3), lambda s: (s, 0, 0)),
            pl.BlockSpec((1, 3, _NPER), lambda s: (s, 0, 0)),
            pl.BlockSpec((1, _NPER, 1), lambda s: (s, 0, 0)),
            pl.BlockSpec((1, 1, 1), lambda s: (s, 0, 0)),
            full((_NZ, _F)),
            full((_LAYERS, 2 * _F + 1, _F)),
            full((_LAYERS, 1, _F)),
            full((_LAYERS, 2 * _F, _F)),
            full((_LAYERS, 1, _F)),
        ],
        out_specs=pl.BlockSpec((1, 1, _F), lambda s: (s, 0, 0)),
        out_shape=jax.ShapeDtypeStruct((_B, 1, _F), f32),
    )(xb, xr, z3, inv, emb, Wm, bm, Wu, bu)
    latent = lat.reshape(_B, _F)

    dims = [_F, 128, 128, 128, 128, 6]
    mWs = [mW0, mW1, mW2, mW3, mW4]
    mbs = [mb0, mb1, mb2, mb3, mb4]
    ops = [latent]
    for i in range(5):
        ops += [mWs[i], mbs[i].reshape(1, dims[i + 1])]
    out = pl.pallas_call(
        _mlp_body,
        out_shape=jax.ShapeDtypeStruct((_B, 6), f32),
    )(*ops)
    return out[:, :3], out[:, 3:]


# speedup vs baseline: 13.9252x; 1.4494x over previous
import jax
import jax.numpy as jnp
from jax.experimental import pallas as pl

_B = 64
_NPER = 128
_N = _B * _NPER
_F = 256
_KNN = 16
_LAYERS = 3
_NZ = 100


def _dot(a, b, prec=jax.lax.Precision.HIGHEST):
    return jnp.dot(a, b, preferred_element_type=jnp.float32, precision=prec)


def _dot1(a, b):
    return jnp.dot(a, b, preferred_element_type=jnp.float32,
                   precision=jax.lax.Precision.DEFAULT)


def _hilo(x):
    hi = x.astype(jnp.bfloat16)
    lo = (x - hi.astype(jnp.float32)).astype(jnp.bfloat16)
    return hi, lo


def _dot3(a, bhi, blo):
    ahi, alo = _hilo(a)
    return _dot1(ahi, bhi) + _dot1(ahi, blo) + _dot1(alo, bhi)


def _mpnn_body(xc_ref, xr_ref, z_ref, inv_ref, ehi_ref, elo_ref, Wch_ref,
               Wcl_ref, wd_ref, bm_ref, Wuh_ref, Wul_ref, bu_ref, lat_ref):
    f32 = jnp.float32
    xc = xc_ref[0]
    xr = xr_ref[0]
    fc = jnp.mod(xc, 1.0)
    fr = jnp.mod(xr, 1.0)

    ri = jax.lax.broadcasted_iota(jnp.int32, (_NPER, _NPER), 0)
    ci = jax.lax.broadcasted_iota(jnp.int32, (_NPER, _NPER), 1)
    dist2 = jnp.where(ri == ci, 1e9, 0.0).astype(f32)
    for c in range(3):
        dc = fc[:, c:c + 1] - fr[c:c + 1, :]
        dc = dc - jnp.round(dc)
        dist2 = dist2 + dc * dc

    key = (jax.lax.bitcast_convert_type(dist2, jnp.int32) & (-128)) | ci
    oh_rows = []
    dists = []
    for _ in range(_KNN):
        m = jnp.min(key, axis=1, keepdims=True)
        idx = m & 127
        oh_rows.append((idx == ci).astype(jnp.bfloat16))
        d2 = jax.lax.bitcast_convert_type(m & (-128), f32)
        dists.append(jnp.sqrt(d2 + 1e-12))
        key = jnp.where(ci == idx, jnp.iinfo(jnp.int32).max, key)
    ohs = jnp.concatenate(oh_rows, axis=0)

    zc = z_ref[0]
    zoh = (zc == jax.lax.broadcasted_iota(jnp.int32, (_NPER, _NZ), 1)
           ).astype(jnp.bfloat16)
    h = _dot1(zoh, ehi_ref[...]) + _dot1(zoh, elo_ref[...])

    for l in range(_LAYERS):
        AA = _dot3(h, Wch_ref[l], Wcl_ref[l])
        Adst = AA[:, :_F] + bm_ref[l]
        hi, lo = _hilo(AA[:, _F:])
        G = _dot1(ohs, hi) + _dot1(ohs, lo)
        wd = wd_ref[l]
        acc = jnp.zeros((_NPER, _F), dtype=f32)
        for k in range(_KNN):
            acc = acc + jax.nn.relu(
                G[k * _NPER:(k + 1) * _NPER] + Adst + dists[k] * wd)
        agg = acc * (1.0 / _KNN)
        u = _dot3(jnp.concatenate([h, agg], axis=1),
                  Wuh_ref[l], Wul_ref[l]) + bu_ref[l]
        h = h + jax.nn.relu(u)

    lat_ref[0] = jnp.sum(h, axis=0, keepdims=True) * inv_ref[0]


def _mlp_body(lat_ref, w0, b0, w1, b1, w2, b2, w3, b3, w4, b4, out_ref):
    t = lat_ref[...]
    for i, (w, b) in enumerate(((w0, b0), (w1, b1), (w2, b2),
                                (w3, b3), (w4, b4))):
        t = _dot(t, w[...], jax.lax.Precision.HIGHEST) + b[...]
        if i < 4:
            t = jax.nn.relu(t)
    out_ref[...] = t


def kernel(cell, x, z, struct_size, emb, Wm0, bm0, Wm1, bm1, Wm2, bm2,
           Wu0, bu0, Wu1, bu1, Wu2, bu2, mW0, mb0, mW1, mb1, mW2, mb2,
           mW3, mb3, mW4, mb4):
    f32 = jnp.float32
    xb = x.reshape(_B, _NPER, 3)
    xr = xb.transpose(0, 2, 1)
    z3 = z.reshape(_B, _NPER, 1)
    inv = (1.0 / struct_size.astype(f32)).reshape(_B, 1, 1)
    ehi = emb.astype(jnp.bfloat16)
    elo = (emb - ehi.astype(f32)).astype(jnp.bfloat16)
    Wms = [Wm0, Wm1, Wm2]
    Wc = jnp.stack([jnp.concatenate([w[:_F], w[_F:2 * _F]], axis=1)
                    for w in Wms])
    Wch = Wc.astype(jnp.bfloat16)
    Wcl = (Wc - Wch.astype(f32)).astype(jnp.bfloat16)
    wd = jnp.stack([w[2 * _F:2 * _F + 1] for w in Wms])
    bm = jnp.stack([bm0, bm1, bm2]).reshape(_LAYERS, 1, _F)
    Wu = jnp.stack([Wu0, Wu1, Wu2])
    Wuh = Wu.astype(jnp.bfloat16)
    Wul = (Wu - Wuh.astype(f32)).astype(jnp.bfloat16)
    bu = jnp.stack([bu0, bu1, bu2]).reshape(_LAYERS, 1, _F)

    full = lambda shp: pl.BlockSpec(shp, lambda s: (0,) * len(shp))
    lat = pl.pallas_call(
        _mpnn_body,
        grid=(_B,),
        in_specs=[
            pl.BlockSpec((1, _NPER, 3), lambda s: (s, 0, 0)),
            pl.BlockSpec((1, 3, _NPER), lambda s: (s, 0, 0)),
            pl.BlockSpec((1, _NPER, 1), lambda s: (s, 0, 0)),
            pl.BlockSpec((1, 1, 1), lambda s: (s, 0, 0)),
            full((_NZ, _F)),
            full((_NZ, _F)),
            full((_LAYERS, _F, 2 * _F)),
            full((_LAYERS, _F, 2 * _F)),
            full((_LAYERS, 1, _F)),
            full((_LAYERS, 1, _F)),
            full((_LAYERS, 2 * _F, _F)),
            full((_LAYERS, 2 * _F, _F)),
            full((_LAYERS, 1, _F)),
        ],
        out_specs=pl.BlockSpec((1, 1, _F), lambda s: (s, 0, 0)),
        out_shape=jax.ShapeDtypeStruct((_B, 1, _F), f32),
    )(xb, xr, z3, inv, ehi, elo, Wch, Wcl, wd, bm, Wuh, Wul, bu)
    latent = lat.reshape(_B, _F)

    dims = [_F, 128, 128, 128, 128, 6]
    mWs = [mW0, mW1, mW2, mW3, mW4]
    mbs = [mb0, mb1, mb2, mb3, mb4]
    ops = [latent]
    for i in range(5):
        ops += [mWs[i], mbs[i].reshape(1, dims[i + 1])]
    out = pl.pallas_call(
        _mlp_body,
        out_shape=jax.ShapeDtypeStruct((_B, 6), f32),
    )(*ops)
    return out[:, :3], out[:, 3:]
